# baseline (speedup 1.0000x reference)
import functools
import jax
import jax.numpy as jnp
from jax import lax
from jax.experimental import pallas as pl
from jax.experimental.pallas import tpu as pltpu
from jax.experimental.pallas import tpu_sc as plsc

VOCAB = 100000
EMB = 64
REGION = 5
BATCH = 1024
SEQ = 200

NC = 2
NS = 16
NW = NC * NS
ROWS_PER_W = BATCH // NW
LANES = 16
GROUPS = EMB // LANES
CHUNK0 = 128
CHUNK1 = SEQ - CHUNK0
WOFF = 8
WROWS = 216


def _sc_body(seq_hbm, semb_hbm, u_hbm, out_hbm, idx_v, rows_v, win_v, out_v, sem):
    c = lax.axis_index("c")
    s = lax.axis_index("s")
    wid = s * NC + c

    zeros = jnp.zeros((LANES,), jnp.float32)
    for row in (WOFF - 2, WOFF - 1, WOFF + SEQ, WOFF + SEQ + 1):
        for g in range(GROUPS):
            win_v[row, pl.ds(g * LANES, LANES)] = zeros

    def row_body(i, carry):
        b = wid * ROWS_PER_W + i
        pltpu.sync_copy(seq_hbm.at[b], idx_v)
        cp0 = pltpu.async_copy(
            u_hbm.at[idx_v.at[pl.ds(0, CHUNK0)]], rows_v.at[pl.ds(0, CHUNK0)], sem)
        cp1 = pltpu.async_copy(
            u_hbm.at[idx_v.at[pl.ds(CHUNK0, CHUNK1)]],
            rows_v.at[pl.ds(CHUNK0, CHUNK1)], sem)
        pltpu.sync_copy(semb_hbm.at[b], win_v.at[pl.ds(WOFF, SEQ)])
        cp0.wait()
        cp1.wait()

        def tok(l, carry2):
            for g in range(GROUPS):
                sl = pl.ds(g * LANES, LANES)
                base = l + (WOFF - 2)
                acc = win_v[base, sl] * rows_v[l, pl.ds(g * LANES, LANES)]
                for r in range(1, REGION):
                    w = win_v[base + r, sl]
                    u = rows_v[l, pl.ds(r * EMB + g * LANES, LANES)]
                    acc = jnp.maximum(acc, w * u)
                out_v[l, sl] = acc
            return carry2

        lax.fori_loop(0, SEQ, tok, 0, unroll=2)
        pltpu.sync_copy(out_v, out_hbm.at[b])
        return carry

    lax.fori_loop(0, ROWS_PER_W, row_body, 0)


@jax.jit
def _region_embed(seq, seq_emb, U):
    u2 = U.reshape(VOCAB, REGION * EMB)
    seq2 = seq.astype(jnp.int32)
    mesh = plsc.VectorSubcoreMesh(core_axis_name="c", subcore_axis_name="s")
    f = pl.kernel(
        _sc_body,
        out_type=jax.ShapeDtypeStruct((BATCH, SEQ, EMB), jnp.float32),
        mesh=mesh,
        scratch_types=[
            pltpu.VMEM((SEQ,), jnp.int32),
            pltpu.VMEM((SEQ, REGION * EMB), jnp.float32),
            pltpu.VMEM((WROWS, EMB), jnp.float32),
            pltpu.VMEM((SEQ, EMB), jnp.float32),
            pltpu.SemaphoreType.DMA,
        ],
        compiler_params=pltpu.CompilerParams(use_tc_tiling_on_sc=False),
    )
    return f(seq2, seq_emb, u2)


def kernel(seq, seq_emb, U):
    return _region_embed(seq, seq_emb, U)
